# baseline (device time: 305123 ns/iter reference)
import functools

import jax
import jax.numpy as jnp
from jax import lax
from jax.experimental import pallas as pl
from jax.experimental.pallas import tpu as pltpu

N_DEV = 32
B, Sq, Skv = 2, 512, 512
H_PER, Dh = 8, 64
D_LOC = H_PER * Dh
D_MODEL = 768
ROWS = B * Sq
CHUNK = ROWS // N_DEV
N_STEP = N_DEV - 1


def _body(x_ref, wq_ref, k_ref, v_ref, wo_ref, out_ref,
          recv_ref, send_sems, recv_sems):
    my = lax.axis_index("i")
    left = lax.rem(my + N_DEV - 1, N_DEV)
    right = lax.rem(my + 1, N_DEV)

    barrier = pltpu.get_barrier_semaphore()
    for nbr in (left, right):
        pl.semaphore_signal(barrier, inc=1, device_id=(nbr,),
                            device_id_type=pl.DeviceIdType.MESH)
    pl.semaphore_wait(barrier, 2)

    q = jnp.dot(x_ref[...], wq_ref[...],
                preferred_element_type=jnp.float32)

    qi = lax.broadcasted_iota(jnp.int32, (Sq, Skv), 0)
    ki = lax.broadcasted_iota(jnp.int32, (Sq, Skv), 1)
    mask = (jnp.abs(qi - ki) <= 128) | (ki < 32) | (qi < 32)
    neg = jnp.float32(-1e9)

    for b in range(B):
        ctx_parts = []
        for h in range(H_PER):
            qbh = q[b * Sq:(b + 1) * Sq, h * Dh:(h + 1) * Dh]
            kbh = k_ref[b * Skv:(b + 1) * Skv, h * Dh:(h + 1) * Dh]
            vbh = v_ref[b * Skv:(b + 1) * Skv, h * Dh:(h + 1) * Dh]
            s = lax.dot_general(qbh, kbh, (((1,), (1,)), ((), ())),
                                preferred_element_type=jnp.float32) * 0.125
            s = jnp.where(mask, s, neg)
            s = s - jnp.max(s, axis=1, keepdims=True)
            w = jnp.exp(s)
            w = w / jnp.sum(w, axis=1, keepdims=True)
            ctx_parts.append(jnp.dot(w, vbh,
                                     preferred_element_type=jnp.float32))
        ctx_b = jnp.concatenate(ctx_parts, axis=1)
        out_ref[b * Sq:(b + 1) * Sq, :] = jnp.dot(
            ctx_b, wo_ref[...], preferred_element_type=jnp.float32)


    for s in range(N_STEP):
        send_idx = lax.rem(my - s + 2 * N_DEV, N_DEV)
        recv_idx = lax.rem(my - s - 1 + 2 * N_DEV, N_DEV)
        rdma = pltpu.make_async_remote_copy(
            src_ref=out_ref.at[pl.ds(send_idx * CHUNK, CHUNK), :],
            dst_ref=recv_ref.at[s],
            send_sem=send_sems.at[s],
            recv_sem=recv_sems.at[s],
            device_id=(right,),
            device_id_type=pl.DeviceIdType.MESH,
        )
        rdma.start()
        rdma.wait()
        rows = pl.ds(recv_idx * CHUNK, CHUNK)
        out_ref[rows, :] = out_ref[rows, :] + recv_ref[s]

    for s in range(N_STEP):
        t = N_STEP + s
        send_idx = lax.rem(my + 1 - s + 2 * N_DEV, N_DEV)
        rdma = pltpu.make_async_remote_copy(
            src_ref=out_ref.at[pl.ds(send_idx * CHUNK, CHUNK), :],
            dst_ref=out_ref.at[pl.ds(send_idx * CHUNK, CHUNK), :],
            send_sem=send_sems.at[t],
            recv_sem=recv_sems.at[t],
            device_id=(right,),
            device_id_type=pl.DeviceIdType.MESH,
        )
        rdma.start()
        rdma.wait()

    @functools.partial(pl.run_scoped, done_sem=pltpu.SemaphoreType.REGULAR)
    def _(done_sem):
        pl.semaphore_signal(done_sem, inc=1, device_id=(left,),
                            device_id_type=pl.DeviceIdType.MESH)
        pl.semaphore_wait(done_sem, 1)


def kernel(x, Wq, K_ext, V_ext, Wo):
    my = lax.axis_index("i")
    x2 = x.reshape(ROWS, D_MODEL)
    k2 = lax.dynamic_slice_in_dim(K_ext, my * H_PER, H_PER, axis=2)
    v2 = lax.dynamic_slice_in_dim(V_ext, my * H_PER, H_PER, axis=2)
    k2 = k2.reshape(B * Skv, D_LOC)
    v2 = v2.reshape(B * Skv, D_LOC)

    out2 = pl.pallas_call(
        _body,
        out_shape=jax.ShapeDtypeStruct((ROWS, D_MODEL), jnp.float32),
        in_specs=[pl.BlockSpec(memory_space=pltpu.VMEM)] * 5,
        out_specs=pl.BlockSpec(memory_space=pltpu.VMEM),
        scratch_shapes=[
            pltpu.VMEM((N_STEP, CHUNK, D_MODEL), jnp.float32),
            pltpu.SemaphoreType.DMA((2 * N_STEP,)),
            pltpu.SemaphoreType.DMA((2 * N_STEP,)),
        ],
        compiler_params=pltpu.CompilerParams(collective_id=0),
    )(x2, Wq, k2, v2, Wo)
    return out2.reshape(B, Sq, D_MODEL)


# device time: 287894 ns/iter; 1.0598x vs baseline; 1.0598x over previous
import jax
import jax.numpy as jnp
from jax import lax
from jax.experimental import pallas as pl
from jax.experimental.pallas import tpu as pltpu

N_DEV = 32
B, Sq, Skv = 2, 512, 512
H_PER, Dh = 8, 64
D_LOC = H_PER * Dh
D_MODEL = 768
ROWS = B * Sq
CHUNK = ROWS // N_DEV
CHUNKS_PER_B = Sq // CHUNK


def _body(x_ref, wq_ref, wo_ref, k_hbm, v_hbm, out_ref,
          k_scr, v_scr, rs_recv,
          kv_sems, rs_send_sems, rs_recv_sems, ag_send_sems, ag_recv_sems):
    my = lax.axis_index("i")

    barrier = pltpu.get_barrier_semaphore()
    for j in range(1, N_DEV):
        pl.semaphore_signal(barrier, inc=1,
                            device_id=(lax.rem(my + j, N_DEV),),
                            device_id_type=pl.DeviceIdType.MESH)

    kv_copies = []
    for t, (hbm, scr) in enumerate(((k_hbm, k_scr), (v_hbm, v_scr))):
        for b in range(B):
            for h in range(H_PER):
                c = pltpu.make_async_copy(
                    hbm.at[b, :, my * H_PER + h, :],
                    scr.at[b, h],
                    kv_sems.at[t, b, h],
                )
                c.start()
                kv_copies.append(c)

    q = jnp.dot(x_ref[...], wq_ref[...],
                preferred_element_type=jnp.float32)

    qi = lax.broadcasted_iota(jnp.int32, (Sq, Skv), 0)
    ki = lax.broadcasted_iota(jnp.int32, (Sq, Skv), 1)
    mask = (jnp.abs(qi - ki) <= 128) | (ki < 32) | (qi < 32)
    neg = jnp.float32(-1e9)

    for c in kv_copies:
        c.wait()
    pl.semaphore_wait(barrier, N_DEV - 1)

    for b in range(B):
        ctx_parts = []
        for h in range(H_PER):
            qbh = q[b * Sq:(b + 1) * Sq, h * Dh:(h + 1) * Dh]
            s = lax.dot_general(qbh, k_scr[b, h], (((1,), (1,)), ((), ())),
                                preferred_element_type=jnp.float32) * 0.125
            s = jnp.where(mask, s, neg)
            s = s - jnp.max(s, axis=1, keepdims=True)
            w = jnp.exp(s)
            w = w / jnp.sum(w, axis=1, keepdims=True)
            ctx_parts.append(jnp.dot(w, v_scr[b, h],
                                     preferred_element_type=jnp.float32))
        ctx_b = jnp.concatenate(ctx_parts, axis=1)
        out_ref[b * Sq:(b + 1) * Sq, :] = jnp.dot(
            ctx_b, wo_ref[...], preferred_element_type=jnp.float32)

        for d in range(b * CHUNKS_PER_B, (b + 1) * CHUNKS_PER_B):
            @pl.when(d != my)
            def _(d=d):
                off = lax.rem(my - d + N_DEV, N_DEV)
                rdma = pltpu.make_async_remote_copy(
                    src_ref=out_ref.at[pl.ds(d * CHUNK, CHUNK), :],
                    dst_ref=rs_recv.at[off],
                    send_sem=rs_send_sems.at[d],
                    recv_sem=rs_recv_sems.at[off],
                    device_id=(d,),
                    device_id_type=pl.DeviceIdType.MESH,
                )
                rdma.start()

    for j in range(1, N_DEV):
        pltpu.make_async_remote_copy(
            src_ref=rs_recv.at[j], dst_ref=rs_recv.at[j],
            send_sem=rs_send_sems.at[0], recv_sem=rs_recv_sems.at[j],
            device_id=(my,), device_id_type=pl.DeviceIdType.MESH,
        ).wait_recv()
    mine = pl.ds(my * CHUNK, CHUNK)
    out_ref[mine, :] = out_ref[mine, :] + jnp.sum(rs_recv[1:], axis=0)

    for d in range(N_DEV):
        @pl.when(d != my)
        def _(d=d):
            rdma = pltpu.make_async_remote_copy(
                src_ref=out_ref.at[mine, :],
                dst_ref=out_ref.at[mine, :],
                send_sem=ag_send_sems.at[d],
                recv_sem=ag_recv_sems.at[my],
                device_id=(d,),
                device_id_type=pl.DeviceIdType.MESH,
            )
            rdma.start()

    for j in range(N_DEV):
        @pl.when(j != my)
        def _(j=j):
            pltpu.make_async_remote_copy(
                src_ref=out_ref.at[pl.ds(j * CHUNK, CHUNK), :],
                dst_ref=out_ref.at[pl.ds(j * CHUNK, CHUNK), :],
                send_sem=ag_send_sems.at[j],
                recv_sem=ag_recv_sems.at[j],
                device_id=(my,), device_id_type=pl.DeviceIdType.MESH,
            ).wait_recv()

    for d in range(N_DEV):
        @pl.when(d != my)
        def _(d=d):
            pltpu.make_async_remote_copy(
                src_ref=out_ref.at[pl.ds(d * CHUNK, CHUNK), :],
                dst_ref=rs_recv.at[0],
                send_sem=rs_send_sems.at[d],
                recv_sem=rs_recv_sems.at[0],
                device_id=(d,), device_id_type=pl.DeviceIdType.MESH,
            ).wait_send()
            pltpu.make_async_remote_copy(
                src_ref=out_ref.at[mine, :],
                dst_ref=rs_recv.at[0],
                send_sem=ag_send_sems.at[d],
                recv_sem=rs_recv_sems.at[0],
                device_id=(d,), device_id_type=pl.DeviceIdType.MESH,
            ).wait_send()


def kernel(x, Wq, K_ext, V_ext, Wo):
    x2 = x.reshape(ROWS, D_MODEL)
    out2 = pl.pallas_call(
        _body,
        out_shape=jax.ShapeDtypeStruct((ROWS, D_MODEL), jnp.float32),
        in_specs=[
            pl.BlockSpec(memory_space=pltpu.VMEM),
            pl.BlockSpec(memory_space=pltpu.VMEM),
            pl.BlockSpec(memory_space=pltpu.VMEM),
            pl.BlockSpec(memory_space=pltpu.MemorySpace.HBM),
            pl.BlockSpec(memory_space=pltpu.MemorySpace.HBM),
        ],
        out_specs=pl.BlockSpec(memory_space=pltpu.VMEM),
        scratch_shapes=[
            pltpu.VMEM((B, H_PER, Skv, Dh), jnp.float32),
            pltpu.VMEM((B, H_PER, Skv, Dh), jnp.float32),
            pltpu.VMEM((N_DEV, CHUNK, D_MODEL), jnp.float32),
            pltpu.SemaphoreType.DMA((2, B, H_PER)),
            pltpu.SemaphoreType.DMA((N_DEV,)),
            pltpu.SemaphoreType.DMA((N_DEV,)),
            pltpu.SemaphoreType.DMA((N_DEV,)),
            pltpu.SemaphoreType.DMA((N_DEV,)),
        ],
        compiler_params=pltpu.CompilerParams(collective_id=0),
    )(x2, Wq, Wo, K_ext, V_ext)
    return out2.reshape(B, Sq, D_MODEL)
